# baseline (speedup 1.0000x reference)
import functools

import jax
import jax.numpy as jnp
from jax import lax
from jax.experimental import pallas as pl
from jax.experimental.pallas import tpu as pltpu
from jax.experimental.pallas import tpu_sc as plsc

VOCAB = 1000000
EMB = 64
BATCH = 16384

_INFO = plsc.get_sparse_core_info()
_NC = _INFO.num_cores
_NS = _INFO.num_subcores
_NW = _NC * _NS
_L = 16
_BPW = BATCH // _NW

_NWIN = 7812
_WPW = 245
_TAIL0 = _NWIN * 128
_RPW = 2 * _BPW
_RCAP = 36448
_SEG = _RPW
_WB = 4
_NBLK = (_WPW + _WB - 1) // _WB


def _iota():
    return lax.iota(jnp.int32, _L)


def _bc(s):
    return jnp.full((_L,), s, jnp.int32)


def _scal(ref, idx_scalar):
    return plsc.load_gather(ref, [_bc(idx_scalar)])[0]


def _rsqrt_newton(x):
    i = plsc.bitcast(x, jnp.int32)
    magic = jnp.full((_L,), 0x5F3759DF, jnp.int32)
    y = plsc.bitcast(magic - lax.shift_right_logical(i, 1), jnp.float32)
    for _ in range(3):
        y = y * (1.5 - 0.5 * x * y * y)
    return y


def _sort_runs(key):
    iota = _iota()
    ks, perm = plsc.sort_key_val(key, iota)
    prev = ks.at[jnp.maximum(iota - 1, 0)].get(mode="promise_in_bounds")
    nxt = ks.at[jnp.minimum(iota + 1, _L - 1)].get(mode="promise_in_bounds")
    neq_prev = (ks != prev) | (iota == 0)
    last = (ks != nxt) | (iota == _L - 1)
    run_start = plsc.cummax(jnp.where(neq_prev, iota, 0))
    run_pos = iota - run_start
    return ks, perm, run_pos, last


def _route_body(x0_hbm, x1_hbm, rbins_hbm, cnts_hbm,
                idx_v, bin_v, cnt_v, sem):
    wid = lax.axis_index("s") * _NC + lax.axis_index("c")
    base = wid * _BPW

    pltpu.sync_copy(x0_hbm.at[pl.ds(base, _BPW)], idx_v.at[pl.ds(0, _BPW)])
    pltpu.sync_copy(x1_hbm.at[pl.ds(base, _BPW)],
                    idx_v.at[pl.ds(_BPW, _BPW)])

    zeros = jnp.zeros((_L,), jnp.int32)
    for b in range(32 // _L):
        cnt_v[pl.ds(b * _L, _L)] = zeros

    def chunk(c, _):
        v = idx_v[pl.ds(c * _L, _L)]
        wg = lax.shift_right_logical(v, 7)
        dst = wg // _WPW
        wl = wg - dst * _WPW
        i = c * _L + _iota()
        f = jnp.where(i < _BPW, base + i, BATCH + base + (i - _BPW))
        rec = lax.shift_left(wl, 22) | lax.shift_left(f, 7) | (v & 127)
        ks, perm, run_pos, last = _sort_runs(dst)
        recs = rec.at[perm].get(mode="promise_in_bounds")
        cbase = plsc.load_gather(cnt_v, [ks])
        pos = cbase + run_pos
        plsc.store_scatter(bin_v, [ks * _SEG + pos], recs)
        plsc.store_scatter(cnt_v, [ks], pos + 1, mask=last)
        return 0

    lax.fori_loop(0, _RPW // _L, chunk, 0)

    pltpu.sync_copy(bin_v, rbins_hbm.at[pl.ds(wid * _NW * _SEG,
                                              _NW * _SEG)])
    pltpu.sync_copy(cnt_v, cnts_hbm.at[pl.ds(wid * _NW, _NW)])


@functools.partial(
    pl.kernel,
    out_type=(jax.ShapeDtypeStruct((_NW * _NW * _SEG,), jnp.int32),
              jax.ShapeDtypeStruct((_NW * _NW,), jnp.int32)),
    mesh=plsc.VectorSubcoreMesh(core_axis_name="c", subcore_axis_name="s"),
    scratch_types=[
        pltpu.VMEM((_RPW,), jnp.int32),
        pltpu.VMEM((_NW * _SEG,), jnp.int32),
        pltpu.VMEM((32,), jnp.int32),
        pltpu.SemaphoreType.DMA,
    ],
    compiler_params=pltpu.CompilerParams(needs_layout_passes=False),
)
def _route(x0_hbm, x1_hbm, rbins_hbm, cnts_hbm, *rest):
    _route_body(x0_hbm, x1_hbm, rbins_hbm, cnts_hbm, *rest)


def _extract_body(tab_hbm, tail_hbm, rbins_hbm, cnts_hbm, stage_hbm,
                  rin_v, cnts_v, cnt_v, off_v, rec_v, buf3, btail, ring,
                  sem_f, sem_o):
    wid = lax.axis_index("s") * _NC + lax.axis_index("c")
    w0 = wid * _WPW

    pltpu.sync_copy(cnts_hbm, cnts_v)

    zeros = jnp.zeros((_L,), jnp.int32)
    for b in range(256 // _L):
        cnt_v[pl.ds(b * _L, _L)] = zeros
        off_v[pl.ds(b * _L, _L)] = zeros

    def seg_loop(pass_b):
        for half in range(2):
            for s16 in range(_NW // 2):
                s = half * (_NW // 2) + s16
                pltpu.async_copy(
                    rbins_hbm.at[pl.ds((s * _NW + wid) * _SEG, _SEG)],
                    rin_v.at[pl.ds(s16 * _SEG, _SEG)], sem_f)
            for s16 in range(_NW // 2):
                pltpu.make_async_copy(
                    rbins_hbm.at[pl.ds(0, _SEG)],
                    rin_v.at[pl.ds(s16 * _SEG, _SEG)], sem_f).wait()

            def per_seg(s16, _):
                n = _scal(cnts_v, (half * (_NW // 2) + s16) * _NW + wid)

                def per_chunk(i, _):
                    off = s16 * _SEG + i * _L
                    recs = rin_v[pl.ds(off, _L)]
                    live = (i * _L + _iota()) < n
                    wl = lax.shift_right_logical(recs, 22)
                    key = jnp.where(live, wl, 255)
                    ks, perm, run_pos, last = _sort_runs(key)
                    base = plsc.load_gather(cnt_v, [ks])
                    pos = base + run_pos
                    if pass_b:
                        rs = recs.at[perm].get(mode="promise_in_bounds")
                        plsc.store_scatter(rec_v, [pos], rs, mask=ks != 255)
                    plsc.store_scatter(cnt_v, [ks], pos + 1,
                                       mask=last & (ks != 255))
                    return 0

                return lax.fori_loop(0, (n + _L - 1) // _L, per_chunk, 0)

            lax.fori_loop(0, _NW // 2, per_seg, 0)

    seg_loop(False)

    def prefix(wl, run):
        n = _scal(cnt_v, wl)
        plsc.store_scatter(off_v, [_bc(wl)], _bc(run), mask=_iota() == 0)
        plsc.store_scatter(cnt_v, [_bc(wl)], _bc(run), mask=_iota() == 0)
        return (run + n + _L - 1) & ~(_L - 1)

    lax.fori_loop(0, _WPW, prefix, 0)

    seg_loop(True)

    @pl.when(w0 + _WPW > _NWIN)
    def _():
        pltpu.sync_copy(tail_hbm, btail)

    def fetch(b, parity):
        tb = w0 + b * _WB

        @pl.when((b < _NBLK) & (tb + _WB <= _NWIN))
        def _():
            pltpu.async_copy(
                tab_hbm.at[:, pl.ds(tb * 128, _WB * 128)],
                buf3.at[parity], sem_f)

        @pl.when((b < _NBLK) & (tb + _WB > _NWIN))
        def _():
            for q in range(_WB):
                @pl.when(tb + q < _NWIN)
                def _():
                    pltpu.async_copy(
                        tab_hbm.at[:, pl.ds((tb + q) * 128, 128)],
                        buf3.at[parity, :, pl.ds(q * 128, 128)], sem_f)

    def fetch_wait(b, parity):
        tb = w0 + b * _WB

        @pl.when(tb + _WB <= _NWIN)
        def _():
            pltpu.make_async_copy(tab_hbm.at[:, pl.ds(0, _WB * 128)],
                                  buf3.at[parity], sem_f).wait()

        @pl.when(tb + _WB > _NWIN)
        def _():
            for q in range(_WB):
                @pl.when(tb + q < _NWIN)
                def _():
                    pltpu.make_async_copy(
                        tab_hbm.at[:, pl.ds(0, 128)],
                        buf3.at[parity, :, pl.ds(q * 128, 128)],
                        sem_f).wait()

    fetch(0, 0)

    def one_record(rec, live, parity, q, is_tail, rc):
        def issue(rc):
            lane = rec & 127
            f = lax.shift_right_logical(rec, 7) & (2 * BATCH - 1)
            rslot = rc & 7
            for c4 in range(EMB // _L):
                kv = c4 * _L + _iota()
                g_main = plsc.load_gather(
                    buf3, [_bc(parity), kv, _bc(q * 128 + lane)])
                g_tail = plsc.load_gather(btail, [kv, _bc(lane & 63)])
                ring[pl.ds(rslot * EMB + c4 * _L, _L)] = jnp.where(
                    _bc(is_tail) > 0, g_tail, g_main)
            pltpu.async_copy(ring.at[pl.ds(rslot * EMB, EMB)],
                             stage_hbm.at[pl.ds(f * EMB, EMB)], sem_o)

            @pl.when(rc >= 8)
            def _():
                pltpu.make_async_copy(stage_hbm.at[pl.ds(0, EMB)],
                                      ring.at[pl.ds(0, EMB)], sem_o).wait()
            return rc + 1

        return lax.cond(live, issue, lambda rc: rc, rc)

    def block(b, rc):
        parity = b & 1
        fetch_wait(b, parity)
        fetch(b + 1, 1 - parity)

        for q in range(_WB):
            t = b * _WB + q
            is_tail = jnp.where(w0 + t == _NWIN, 1, 0)
            start = _scal(off_v, jnp.minimum(t, _WPW - 1))
            end = _scal(cnt_v, jnp.minimum(t, _WPW - 1))
            trips = jnp.where(t < _WPW, (end - start + _L - 1) // _L, 0)

            def do16(i, rc, start=start, end=end, parity=parity, q=q,
                     is_tail=is_tail):
                s16 = start + i * _L
                recv = rec_v[pl.ds(s16, _L)]
                for j in range(_L):
                    rc = one_record(recv[j], s16 + j < end, parity, q,
                                    is_tail, rc)
                return rc

            rc = lax.fori_loop(0, trips, do16, rc)
        return rc

    rc = lax.fori_loop(0, _NBLK, block, 0)

    def drain(_, rcleft):
        @pl.when(rcleft > 0)
        def _():
            pltpu.make_async_copy(stage_hbm.at[pl.ds(0, EMB)],
                                  ring.at[pl.ds(0, EMB)], sem_o).wait()
        return jnp.maximum(rcleft - 1, 0)

    lax.fori_loop(0, 8, drain, jnp.minimum(rc, 8))


@functools.partial(
    pl.kernel,
    out_type=jax.ShapeDtypeStruct((2 * BATCH * EMB,), jnp.float32),
    mesh=plsc.VectorSubcoreMesh(core_axis_name="c", subcore_axis_name="s"),
    scratch_types=[
        pltpu.VMEM((_NW // 2 * _SEG,), jnp.int32),
        pltpu.VMEM((_NW * _NW,), jnp.int32),
        pltpu.VMEM((256,), jnp.int32),
        pltpu.VMEM((256,), jnp.int32),
        pltpu.VMEM((_RCAP,), jnp.int32),
        pltpu.VMEM((2, EMB, _WB * 128), jnp.float32),
        pltpu.VMEM((EMB, EMB), jnp.float32),
        pltpu.VMEM((8 * EMB,), jnp.float32),
        pltpu.SemaphoreType.DMA,
        pltpu.SemaphoreType.DMA,
    ],
    compiler_params=pltpu.CompilerParams(needs_layout_passes=False),
)
def _extract(tab_hbm, tail_hbm, rbins_hbm, cnts_hbm, stage_hbm, *rest):
    _extract_body(tab_hbm, tail_hbm, rbins_hbm, cnts_hbm, stage_hbm, *rest)


def _cosine_body(stage_hbm, out_hbm, t_flat, c_flat, out_v, sem0, sem1):
    wid = lax.axis_index("s") * _NC + lax.axis_index("c")
    base = wid * _BPW

    cp0 = pltpu.async_copy(stage_hbm.at[pl.ds(base * EMB, _BPW * EMB)],
                           t_flat, sem0)
    cp1 = pltpu.async_copy(
        stage_hbm.at[pl.ds((BATCH + base) * EMB, _BPW * EMB)], c_flat, sem1)
    cp0.wait()
    cp1.wait()

    def group(g, _):
        fbase = (g * _L + _iota()) * EMB

        def step(k, carry):
            dot, na, nb = carry
            tv = plsc.load_gather(t_flat, [fbase + k])
            cv = plsc.load_gather(c_flat, [fbase + k])
            return (dot + tv * cv, na + tv * tv, nb + cv * cv)

        zero = jnp.zeros((_L,), jnp.float32)
        dot, na, nb = lax.fori_loop(0, EMB, step, (zero, zero, zero),
                                    unroll=4)
        out_v[pl.ds(g * _L, _L)] = dot * _rsqrt_newton(na * nb)
        return 0

    lax.fori_loop(0, _BPW // _L, group, 0)
    pltpu.sync_copy(out_v, out_hbm.at[pl.ds(base, _BPW)])


@functools.partial(
    pl.kernel,
    out_type=jax.ShapeDtypeStruct((BATCH,), jnp.float32),
    mesh=plsc.VectorSubcoreMesh(core_axis_name="c", subcore_axis_name="s"),
    scratch_types=[
        pltpu.VMEM((_BPW * EMB,), jnp.float32),
        pltpu.VMEM((_BPW * EMB,), jnp.float32),
        pltpu.VMEM((_BPW,), jnp.float32),
        pltpu.SemaphoreType.DMA,
        pltpu.SemaphoreType.DMA,
    ],
    compiler_params=pltpu.CompilerParams(needs_layout_passes=False),
)
def _cosine(stage_hbm, out_hbm, t_flat, c_flat, out_v, sem0, sem1):
    _cosine_body(stage_hbm, out_hbm, t_flat, c_flat, out_v, sem0, sem1)


def kernel(x, table):
    x0 = jnp.asarray(x[:, 0], jnp.int32)
    x1 = jnp.asarray(x[:, 1], jnp.int32)
    table_t = table.T
    tail_t = table_t[:, _TAIL0:]
    rbins, cnts = _route(x0, x1)
    staging = _extract(table_t, tail_t, rbins, cnts)
    out = _cosine(staging)
    return out.reshape(BATCH, 1)


# speedup vs baseline: 1.5302x; 1.5302x over previous
import functools

import jax
import jax.numpy as jnp
from jax import lax
from jax.experimental import pallas as pl
from jax.experimental.pallas import tpu as pltpu
from jax.experimental.pallas import tpu_sc as plsc

VOCAB = 1000000
EMB = 64
BATCH = 16384

_INFO = plsc.get_sparse_core_info()
_NC = _INFO.num_cores
_NS = _INFO.num_subcores
_NW = _NC * _NS
_L = 16
_BPW = BATCH // _NW

_NWIN = 7812
_WPW = 245
_TAIL0 = _NWIN * 128
_RPW = 2 * _BPW
_RCAP = 40960
_SEG = _RPW


def _iota():
    return lax.iota(jnp.int32, _L)


def _bc(s):
    return jnp.full((_L,), s, jnp.int32)


def _scal(ref, idx_scalar):
    return plsc.load_gather(ref, [_bc(idx_scalar)])[0]


def _rsqrt_newton(x):
    i = plsc.bitcast(x, jnp.int32)
    magic = jnp.full((_L,), 0x5F3759DF, jnp.int32)
    y = plsc.bitcast(magic - lax.shift_right_logical(i, 1), jnp.float32)
    for _ in range(3):
        y = y * (1.5 - 0.5 * x * y * y)
    return y


def _sort_runs(key):
    iota = _iota()
    ks, perm = plsc.sort_key_val(key, iota)
    prev = ks.at[jnp.maximum(iota - 1, 0)].get(mode="promise_in_bounds")
    nxt = ks.at[jnp.minimum(iota + 1, _L - 1)].get(mode="promise_in_bounds")
    neq_prev = (ks != prev) | (iota == 0)
    last = (ks != nxt) | (iota == _L - 1)
    run_start = plsc.cummax(jnp.where(neq_prev, iota, 0))
    run_pos = iota - run_start
    return ks, perm, run_pos, last


def _route_body(x0_hbm, x1_hbm, rbins_hbm, cnts_hbm,
                idx_v, bin_v, cnt_v, sem):
    wid = lax.axis_index("s") * _NC + lax.axis_index("c")
    base = wid * _BPW

    pltpu.sync_copy(x0_hbm.at[pl.ds(base, _BPW)], idx_v.at[pl.ds(0, _BPW)])
    pltpu.sync_copy(x1_hbm.at[pl.ds(base, _BPW)],
                    idx_v.at[pl.ds(_BPW, _BPW)])

    zeros = jnp.zeros((_L,), jnp.int32)
    for b in range(32 // _L):
        cnt_v[pl.ds(b * _L, _L)] = zeros

    def chunk(c, _):
        v = idx_v[pl.ds(c * _L, _L)]
        wg = lax.shift_right_logical(v, 7)
        dst = wg // _WPW
        wl = wg - dst * _WPW
        i = c * _L + _iota()
        f = jnp.where(i < _BPW, base + i, BATCH + base + (i - _BPW))
        rec = lax.shift_left(wl, 22) | lax.shift_left(f, 7) | (v & 127)
        ks, perm, run_pos, last = _sort_runs(dst)
        recs = rec.at[perm].get(mode="promise_in_bounds")
        cbase = plsc.load_gather(cnt_v, [ks])
        pos = cbase + run_pos
        plsc.store_scatter(bin_v, [ks * _SEG + pos], recs)
        plsc.store_scatter(cnt_v, [ks], pos + 1, mask=last)
        return 0

    lax.fori_loop(0, _RPW // _L, chunk, 0)

    pltpu.sync_copy(bin_v, rbins_hbm.at[pl.ds(wid * _NW * _SEG,
                                              _NW * _SEG)])
    pltpu.sync_copy(cnt_v, cnts_hbm.at[pl.ds(wid * _NW, _NW)])


@functools.partial(
    pl.kernel,
    out_type=(jax.ShapeDtypeStruct((_NW * _NW * _SEG,), jnp.int32),
              jax.ShapeDtypeStruct((_NW * _NW,), jnp.int32)),
    mesh=plsc.VectorSubcoreMesh(core_axis_name="c", subcore_axis_name="s"),
    scratch_types=[
        pltpu.VMEM((_RPW,), jnp.int32),
        pltpu.VMEM((_NW * _SEG,), jnp.int32),
        pltpu.VMEM((32,), jnp.int32),
        pltpu.SemaphoreType.DMA,
    ],
    compiler_params=pltpu.CompilerParams(needs_layout_passes=False),
)
def _route(x0_hbm, x1_hbm, rbins_hbm, cnts_hbm, *rest):
    _route_body(x0_hbm, x1_hbm, rbins_hbm, cnts_hbm, *rest)


def _extract_body(tab_hbm, tail_hbm, rbins_hbm, cnts_hbm, stage_hbm,
                  rin_v, cnts_v, cnt_v, off_v, rec_v, buf3, btail, ring,
                  sem_f, sem_o):
    wid = lax.axis_index("s") * _NC + lax.axis_index("c")
    w0 = wid * _WPW

    pltpu.sync_copy(cnts_hbm, cnts_v)
    for s in range(_NW):
        pltpu.async_copy(
            rbins_hbm.at[pl.ds((s * _NW + wid) * _SEG, _SEG)],
            rin_v.at[pl.ds(s * _SEG, _SEG)], sem_f)
    for s in range(_NW):
        pltpu.make_async_copy(rbins_hbm.at[pl.ds(0, _SEG)],
                              rin_v.at[pl.ds(s * _SEG, _SEG)], sem_f).wait()

    zeros = jnp.zeros((_L,), jnp.int32)
    for b in range(256 // _L):
        cnt_v[pl.ds(b * _L, _L)] = zeros

    def seg_loop(pass_b):
        def per_seg(s, _):
            n = _scal(cnts_v, s * _NW + wid)

            def per_chunk(i, _):
                off = s * _SEG + i * _L
                recs = rin_v[pl.ds(off, _L)]
                live = (i * _L + _iota()) < n
                wl = lax.shift_right_logical(recs, 22)
                key = jnp.where(live, wl, 255)
                ks, perm, run_pos, last = _sort_runs(key)
                base = plsc.load_gather(cnt_v, [ks])
                pos = base + run_pos
                if pass_b:
                    rs = recs.at[perm].get(mode="promise_in_bounds")
                    plsc.store_scatter(rec_v, [pos], rs, mask=ks != 255)
                plsc.store_scatter(cnt_v, [ks], pos + 1,
                                   mask=last & (ks != 255))
                return 0

            return lax.fori_loop(0, (n + _L - 1) // _L, per_chunk, 0)

        lax.fori_loop(0, _NW, per_seg, 0)

    seg_loop(False)

    def prefix(wl, run):
        n = _scal(cnt_v, wl)
        plsc.store_scatter(off_v, [_bc(wl)], _bc(run), mask=_iota() == 0)
        plsc.store_scatter(cnt_v, [_bc(wl)], _bc(run), mask=_iota() == 0)
        return (run + n + _L - 1) & ~(_L - 1)

    lax.fori_loop(0, _WPW, prefix, 0)

    seg_loop(True)

    @pl.when(w0 + _WPW > _NWIN)
    def _():
        pltpu.sync_copy(tail_hbm, btail)

    def fetch(t, parity):
        @pl.when((t < _WPW) & (w0 + t < _NWIN))
        def _():
            pltpu.async_copy(
                tab_hbm.at[:, pl.ds((w0 + t) * 128, 128)],
                buf3.at[parity], sem_f)

    for p in range(3):
        fetch(p, p)

    def one_record(rec, live, parity, is_tail, rc):
        def issue(rc):
            lane = rec & 127
            f = lax.shift_right_logical(rec, 7) & (2 * BATCH - 1)
            rslot = rc & 7
            for c4 in range(EMB // _L):
                kv = c4 * _L + _iota()
                g_main = plsc.load_gather(buf3, [_bc(parity), kv, _bc(lane)])
                g_tail = plsc.load_gather(btail, [kv, _bc(lane & 63)])
                ring[pl.ds(rslot * EMB + c4 * _L, _L)] = jnp.where(
                    _bc(is_tail) > 0, g_tail, g_main)
            pltpu.async_copy(ring.at[pl.ds(rslot * EMB, EMB)],
                             stage_hbm.at[pl.ds(f * EMB, EMB)], sem_o)

            @pl.when(rc >= 8)
            def _():
                pltpu.make_async_copy(stage_hbm.at[pl.ds(0, EMB)],
                                      ring.at[pl.ds(0, EMB)], sem_o).wait()
            return rc + 1

        return lax.cond(live, issue, lambda rc: rc, rc)

    def window(t, rc):
        parity = t & 3
        is_tail = jnp.where(w0 + t == _NWIN, 1, 0)

        @pl.when(w0 + t < _NWIN)
        def _():
            pltpu.make_async_copy(tab_hbm.at[:, pl.ds(0, 128)],
                                  buf3.at[parity], sem_f).wait()

        fetch(t + 3, (t + 3) & 3)

        start = _scal(off_v, t)
        end = _scal(cnt_v, t)

        def do16(i, rc):
            s16 = start + i * _L
            recv = rec_v[pl.ds(s16, _L)]
            for j in range(_L):
                rc = one_record(recv[j], s16 + j < end, parity, is_tail, rc)
            return rc

        return lax.fori_loop(0, (end - start + _L - 1) // _L, do16, rc)

    rc = lax.fori_loop(0, _WPW, window, 0)

    def drain(_, rcleft):
        @pl.when(rcleft > 0)
        def _():
            pltpu.make_async_copy(stage_hbm.at[pl.ds(0, EMB)],
                                  ring.at[pl.ds(0, EMB)], sem_o).wait()
        return jnp.maximum(rcleft - 1, 0)

    lax.fori_loop(0, 8, drain, jnp.minimum(rc, 8))


@functools.partial(
    pl.kernel,
    out_type=jax.ShapeDtypeStruct((2 * BATCH * EMB,), jnp.float32),
    mesh=plsc.VectorSubcoreMesh(core_axis_name="c", subcore_axis_name="s"),
    scratch_types=[
        pltpu.VMEM((_NW * _SEG,), jnp.int32),
        pltpu.VMEM((_NW * _NW,), jnp.int32),
        pltpu.VMEM((256,), jnp.int32),
        pltpu.VMEM((256,), jnp.int32),
        pltpu.VMEM((_RCAP,), jnp.int32),
        pltpu.VMEM((4, EMB, 128), jnp.float32),
        pltpu.VMEM((EMB, EMB), jnp.float32),
        pltpu.VMEM((8 * EMB,), jnp.float32),
        pltpu.SemaphoreType.DMA,
        pltpu.SemaphoreType.DMA,
    ],
    compiler_params=pltpu.CompilerParams(needs_layout_passes=False),
)
def _extract(tab_hbm, tail_hbm, rbins_hbm, cnts_hbm, stage_hbm, *rest):
    _extract_body(tab_hbm, tail_hbm, rbins_hbm, cnts_hbm, stage_hbm, *rest)


def _cosine_body(stage_hbm, out_hbm, t_flat, c_flat, out_v, sem0, sem1):
    wid = lax.axis_index("s") * _NC + lax.axis_index("c")
    base = wid * _BPW

    cp0 = pltpu.async_copy(stage_hbm.at[pl.ds(base * EMB, _BPW * EMB)],
                           t_flat, sem0)
    cp1 = pltpu.async_copy(
        stage_hbm.at[pl.ds((BATCH + base) * EMB, _BPW * EMB)], c_flat, sem1)
    cp0.wait()
    cp1.wait()

    def group(g, _):
        fbase = (g * _L + _iota()) * EMB

        def step(k, carry):
            dot, na, nb = carry
            tv = plsc.load_gather(t_flat, [fbase + k])
            cv = plsc.load_gather(c_flat, [fbase + k])
            return (dot + tv * cv, na + tv * tv, nb + cv * cv)

        zero = jnp.zeros((_L,), jnp.float32)
        dot, na, nb = lax.fori_loop(0, EMB, step, (zero, zero, zero))
        out_v[pl.ds(g * _L, _L)] = dot * _rsqrt_newton(na * nb)
        return 0

    lax.fori_loop(0, _BPW // _L, group, 0)
    pltpu.sync_copy(out_v, out_hbm.at[pl.ds(base, _BPW)])


@functools.partial(
    pl.kernel,
    out_type=jax.ShapeDtypeStruct((BATCH,), jnp.float32),
    mesh=plsc.VectorSubcoreMesh(core_axis_name="c", subcore_axis_name="s"),
    scratch_types=[
        pltpu.VMEM((_BPW * EMB,), jnp.float32),
        pltpu.VMEM((_BPW * EMB,), jnp.float32),
        pltpu.VMEM((_BPW,), jnp.float32),
        pltpu.SemaphoreType.DMA,
        pltpu.SemaphoreType.DMA,
    ],
    compiler_params=pltpu.CompilerParams(needs_layout_passes=False),
)
def _cosine(stage_hbm, out_hbm, t_flat, c_flat, out_v, sem0, sem1):
    _cosine_body(stage_hbm, out_hbm, t_flat, c_flat, out_v, sem0, sem1)


def kernel(x, table):
    x0 = jnp.asarray(x[:, 0], jnp.int32)
    x1 = jnp.asarray(x[:, 1], jnp.int32)
    table_t = table.T
    tail_t = table_t[:, _TAIL0:]
    rbins, cnts = _route(x0, x1)
    staging = _extract(table_t, tail_t, rbins, cnts)
    out = _cosine(staging)
    return out.reshape(BATCH, 1)
